# baseline (speedup 1.0000x reference)
import dataclasses
import functools

import jax
import jax.numpy as jnp
from jax import lax
from jax.experimental import pallas as pl
from jax.experimental.pallas import tpu as pltpu
from jax.experimental.pallas import tpu_sc as plsc

N = 10000
E = 320000
D = 128

NC = 2
NS = 16
NW = NC * NS
CHUNK = 72
NBUF = 4
IBLK = 4
IBR = 3
PERIOD = 24
NCH = 144
E_PAD = NW * NCH * CHUNK
ROWS_A = 624
ROWS_B = 640


def _sc_compiler_params():
    cp = pltpu.CompilerParams()
    if "needs_layout_passes" in pltpu.CompilerParams.__dataclass_fields__:
        cp = dataclasses.replace(cp, needs_layout_passes=False)
    return cp


def _spmm_sc(h, idx):
    mesh = plsc.VectorSubcoreMesh(core_axis_name="c", subcore_axis_name="s")

    @functools.partial(
        pl.kernel,
        out_type=jax.ShapeDtypeStruct((NC, N, D), jnp.float32),
        mesh=mesh,
        compiler_params=_sc_compiler_params(),
        scratch_types=(
            [pltpu.VMEM((IBLK, 3, CHUNK), jnp.int32)] * IBR
            + [pltpu.VMEM((CHUNK, D), jnp.float32)] * NBUF
            + [pltpu.VMEM_SHARED((N, D), jnp.float32)]
            + [pltpu.SemaphoreType.DMA] * (IBR + 2 * NBUF)),
    )
    def k(h_hbm, idx_hbm, out_hbm, *refs):
        ibufs = refs[:IBR]
        bufs = refs[IBR:IBR + NBUF]
        acc_sh = refs[IBR + NBUF]
        sems = refs[IBR + NBUF + 1:]
        isem = sems[:IBR]
        gsem = sems[IBR:IBR + NBUF]
        ssem = sems[IBR + NBUF:]
        cid = lax.axis_index("c")
        sid = lax.axis_index("s")
        wid = sid * NC + cid

        def i_start(bid, slot):
            pltpu.async_copy(idx_hbm.at[wid].at[bid], ibufs[slot], isem[slot])

        def i_wait(bid, slot):
            pltpu.make_async_copy(
                idx_hbm.at[wid].at[bid], ibufs[slot], isem[slot]).wait()

        def g_start(slot, q, b):
            pltpu.async_copy(
                h_hbm.at[ibufs[slot].at[q].at[0]], bufs[b], gsem[b])

        def g_wait(slot, q, b):
            pltpu.make_async_copy(
                h_hbm.at[ibufs[slot].at[q].at[0]], bufs[b], gsem[b]).wait()

        def s_start(slot, q, b):
            pltpu.async_copy(
                bufs[b], acc_sh.at[ibufs[slot].at[q].at[1]], ssem[b],
                add=True)

        def s_wait(slot, q, b):
            pltpu.make_async_copy(
                bufs[b], acc_sh.at[ibufs[slot].at[q].at[1]], ssem[b]).wait()

        def scale(slot, q, b):
            @plsc.parallel_loop(0, CHUNK, 1, unroll=4)
            def _(e):
                wvi = plsc.load_gather(
                    ibufs[slot].at[q],
                    [jnp.full((16,), 2, jnp.int32),
                     jnp.full((16,), e, jnp.int32)])
                wvec = plsc.bitcast(wvi, jnp.float32)
                for j in range(D // 16):
                    sl = pl.ds(j * 16, 16)
                    bufs[b][e, sl] = bufs[b][e, sl] * wvec

        def step(c, u):
            q = u % IBLK

            @pl.when(c >= 2)
            def _():
                s_wait(((u - 2) // IBLK) % IBR, (u - 2) % IBLK,
                       (u - 2) % NBUF)

            if q == 2:
                @pl.when(c + 2 < NCH)
                def _():
                    i_wait((c + 2) // IBLK, ((u + 2) // IBLK) % IBR)

                @pl.when(c + 6 < NCH)
                def _():
                    i_start((c + 6) // IBLK, ((u + 6) // IBLK) % IBR)

            @pl.when(c + 2 < NCH)
            def _():
                g_start(((u + 2) // IBLK) % IBR, (u + 2) % IBLK,
                        (u + 2) % NBUF)

            g_wait((u // IBLK) % IBR, q, u % NBUF)
            scale((u // IBLK) % IBR, q, u % NBUF)
            s_start((u // IBLK) % IBR, q, u % NBUF)

        i_start(0, 0)
        i_start(1, 1)

        zero = jnp.zeros((16,), jnp.float32)

        @pl.loop(0, CHUNK)
        def _(r):
            for j in range(D // 16):
                bufs[0][r, pl.ds(j * 16, 16)] = zero

        @pl.when(sid < NS - 1)
        def _():
            @pl.loop(0, ROWS_A // 48)
            def _(b):
                pltpu.sync_copy(
                    bufs[0].at[pl.ds(0, 48)],
                    acc_sh.at[pl.ds(sid * ROWS_A + b * 48, 48)])

        @pl.when(sid == NS - 1)
        def _():
            @pl.loop(0, ROWS_B // 64)
            def _(b):
                pltpu.sync_copy(
                    bufs[0].at[pl.ds(0, 64)],
                    acc_sh.at[pl.ds((NS - 1) * ROWS_A + b * 64, 64)])

        plsc.subcore_barrier()

        i_wait(0, 0)
        g_start(0, 0, 0)
        g_start(0, 1, 1)

        @pl.loop(0, NCH // PERIOD)
        def _(r):
            c0 = r * PERIOD
            for u in range(PERIOD):
                step(c0 + u, u)

        for c in (NCH - 2, NCH - 1):
            s_wait((c // IBLK) % IBR, c % IBLK, c % NBUF)

        plsc.subcore_barrier()

        @pl.when(sid < NS - 1)
        def _():
            @pl.loop(0, ROWS_A // 208)
            def _(b):
                r0 = sid * ROWS_A + b * 208
                pltpu.sync_copy(acc_sh.at[pl.ds(r0, 208)],
                                out_hbm.at[cid].at[pl.ds(r0, 208)])

        @pl.when(sid == NS - 1)
        def _():
            @pl.loop(0, ROWS_B // 160)
            def _(b):
                r0 = (NS - 1) * ROWS_A + b * 160
                pltpu.sync_copy(acc_sh.at[pl.ds(r0, 160)],
                                out_hbm.at[cid].at[pl.ds(r0, 160)])

    return k(h, idx)


_BLK = 10000


def _linear_tc(x, W, b):
    def body(x_ref, w_ref, b_ref, o_ref):
        o_ref[...] = lax.dot_general(
            x_ref[...], w_ref[...], (((1,), (1,)), ((), ())),
            preferred_element_type=jnp.float32) + b_ref[...]

    return pl.pallas_call(
        body,
        grid=(N // _BLK,),
        in_specs=[pl.BlockSpec((_BLK, D), lambda i: (i, 0)),
                  pl.BlockSpec((D, D), lambda i: (0, 0)),
                  pl.BlockSpec((1, D), lambda i: (0, 0))],
        out_specs=pl.BlockSpec((_BLK, D), lambda i: (i, 0)),
        out_shape=jax.ShapeDtypeStruct((N, D), jnp.float32),
    )(x, W, b.reshape(1, D))


def _elu_linear_tc(p, W, b):
    def body(p_ref, w_ref, b_ref, o_ref):
        s = p_ref[0] + p_ref[1]
        z = jnp.where(s > 0, s, jnp.exp(s) - 1.0)
        o_ref[...] = lax.dot_general(
            z, w_ref[...], (((1,), (1,)), ((), ())),
            preferred_element_type=jnp.float32) + b_ref[...]

    return pl.pallas_call(
        body,
        grid=(N // _BLK,),
        in_specs=[pl.BlockSpec((NC, _BLK, D), lambda i: (0, i, 0)),
                  pl.BlockSpec((D, D), lambda i: (0, 0)),
                  pl.BlockSpec((1, D), lambda i: (0, 0))],
        out_specs=pl.BlockSpec((_BLK, D), lambda i: (i, 0)),
        out_shape=jax.ShapeDtypeStruct((N, D), jnp.float32),
    )(p, W, b.reshape(1, D))


def _sum2_tc(q):
    def body(q_ref, o_ref):
        o_ref[...] = q_ref[0] + q_ref[1]

    return pl.pallas_call(
        body,
        grid=(N // _BLK,),
        in_specs=[pl.BlockSpec((NC, _BLK, D), lambda i: (0, i, 0))],
        out_specs=pl.BlockSpec((_BLK, D), lambda i: (i, 0)),
        out_shape=jax.ShapeDtypeStruct((N, D), jnp.float32),
    )(q)


def kernel(x, edge_index, edge_weight, W1, b1, W2, b2):
    pad = E_PAD - E
    pad_idx = (jnp.arange(pad, dtype=jnp.int32) * 37) % N
    src = jnp.concatenate([edge_index[1], pad_idx])
    dst = jnp.concatenate([edge_index[0], pad_idx])
    w = jnp.concatenate([edge_weight, jnp.zeros((pad,), jnp.float32)])
    idx = jnp.stack([src.reshape(NW, NCH, CHUNK),
                     dst.reshape(NW, NCH, CHUNK),
                     lax.bitcast_convert_type(w, jnp.int32).reshape(
                         NW, NCH, CHUNK)], axis=2)
    idx = idx.reshape(NW, NCH // IBLK, IBLK, 3, CHUNK)

    h1 = _linear_tc(x, W1, b1)
    p = _spmm_sc(h1, idx)
    h2 = _elu_linear_tc(p, W2, b2)
    q = _spmm_sc(h2, idx)
    return _sum2_tc(q)


# speedup vs baseline: 1.0675x; 1.0675x over previous
import dataclasses
import functools

import jax
import jax.numpy as jnp
from jax import lax
from jax.experimental import pallas as pl
from jax.experimental.pallas import tpu as pltpu
from jax.experimental.pallas import tpu_sc as plsc

N = 10000
E = 320000
D = 128

NC = 2
NS = 16
NW = NC * NS
CHUNK = 80
NBUF = 4
IRING = 8
NCH = 128
E_PAD = NW * NCH * CHUNK
ROWS_A = 624
ROWS_B = 640


def _sc_compiler_params():
    cp = pltpu.CompilerParams()
    if "needs_layout_passes" in pltpu.CompilerParams.__dataclass_fields__:
        cp = dataclasses.replace(cp, needs_layout_passes=False)
    return cp


def _spmm_sc(h, idx):
    mesh = plsc.VectorSubcoreMesh(core_axis_name="c", subcore_axis_name="s")

    @functools.partial(
        pl.kernel,
        out_type=jax.ShapeDtypeStruct((NC, N, D), jnp.float32),
        mesh=mesh,
        compiler_params=_sc_compiler_params(),
        scratch_types=(
            [pltpu.VMEM((3, CHUNK), jnp.int32)] * IRING
            + [pltpu.VMEM((CHUNK, D), jnp.float32)] * NBUF
            + [pltpu.VMEM_SHARED((N, D), jnp.float32)]
            + [pltpu.SemaphoreType.DMA] * (IRING + 2 * NBUF)),
    )
    def k(h_hbm, idx_hbm, out_hbm, *refs):
        ibufs = refs[:IRING]
        bufs = refs[IRING:IRING + NBUF]
        acc_sh = refs[IRING + NBUF]
        sems = refs[IRING + NBUF + 1:]
        isem = sems[:IRING]
        gsem = sems[IRING:IRING + NBUF]
        ssem = sems[IRING + NBUF:]
        cid = lax.axis_index("c")
        sid = lax.axis_index("s")
        wid = sid * NC + cid

        def i_start(c, i):
            pltpu.async_copy(idx_hbm.at[wid].at[c], ibufs[i], isem[i])

        def i_wait(c, i):
            pltpu.make_async_copy(
                idx_hbm.at[wid].at[c], ibufs[i], isem[i]).wait()

        def g_start(c, i, b):
            pltpu.async_copy(h_hbm.at[ibufs[i].at[0]], bufs[b], gsem[b])

        def g_wait(c, i, b):
            pltpu.make_async_copy(
                h_hbm.at[ibufs[i].at[0]], bufs[b], gsem[b]).wait()

        def s_start(c, i, b):
            pltpu.async_copy(
                bufs[b], acc_sh.at[ibufs[i].at[1]], ssem[b], add=True)

        def s_wait(c, i, b):
            pltpu.make_async_copy(
                bufs[b], acc_sh.at[ibufs[i].at[1]], ssem[b]).wait()

        def scale(c, i, b):
            @plsc.parallel_loop(0, CHUNK, 1, unroll=4)
            def _(e):
                wvi = plsc.load_gather(
                    ibufs[i],
                    [jnp.full((16,), 2, jnp.int32),
                     jnp.full((16,), e, jnp.int32)])
                wvec = plsc.bitcast(wvi, jnp.float32)
                for j in range(D // 16):
                    sl = pl.ds(j * 16, 16)
                    bufs[b][e, sl] = bufs[b][e, sl] * wvec

        for c0 in range(NBUF):
            i_start(c0, c0)

        zero = jnp.zeros((16,), jnp.float32)

        @pl.loop(0, CHUNK)
        def _(r):
            for j in range(D // 16):
                bufs[0][r, pl.ds(j * 16, 16)] = zero

        @pl.when(sid < NS - 1)
        def _():
            @pl.loop(0, ROWS_A // 48)
            def _(b):
                pltpu.sync_copy(
                    bufs[0].at[pl.ds(0, 48)],
                    acc_sh.at[pl.ds(sid * ROWS_A + b * 48, 48)])

        @pl.when(sid == NS - 1)
        def _():
            @pl.loop(0, ROWS_B // CHUNK)
            def _(b):
                pltpu.sync_copy(
                    bufs[0],
                    acc_sh.at[pl.ds((NS - 1) * ROWS_A + b * CHUNK, CHUNK)])

        plsc.subcore_barrier()

        i_wait(0, 0)
        g_start(0, 0, 0)
        i_wait(1, 1)
        g_start(1, 1, 1)

        for u in range(IRING):
            c = u
            if c >= 2:
                s_wait(c - 2, (u - 2) % IRING, (u + 2) % NBUF)
            i_wait(c + 2, (u + 2) % IRING)
            g_start(c + 2, (u + 2) % IRING, (u + 2) % NBUF)
            i_start(c + NBUF, (u + NBUF) % IRING)
            g_wait(c, u, u % NBUF)
            scale(c, u, u % NBUF)
            s_start(c, u, u % NBUF)

        @pl.loop(1, NCH // IRING - 1)
        def _(r):
            for u in range(IRING):
                c = r * IRING + u
                s_wait(c - 2, (u - 2) % IRING, (u + 2) % NBUF)
                i_wait(c + 2, (u + 2) % IRING)
                g_start(c + 2, (u + 2) % IRING, (u + 2) % NBUF)
                i_start(c + NBUF, (u + NBUF) % IRING)
                g_wait(c, u, u % NBUF)
                scale(c, u, u % NBUF)
                s_start(c, u, u % NBUF)

        for u in range(IRING):
            c = NCH - IRING + u
            s_wait(c - 2, (u - 2) % IRING, (u + 2) % NBUF)
            if c + 2 < NCH:
                i_wait(c + 2, (u + 2) % IRING)
                g_start(c + 2, (u + 2) % IRING, (u + 2) % NBUF)
            if c + NBUF < NCH:
                i_start(c + NBUF, (u + NBUF) % IRING)
            g_wait(c, u, u % NBUF)
            scale(c, u, u % NBUF)
            s_start(c, u, u % NBUF)

        s_wait(NCH - 2, (NCH - 2) % IRING, (NCH - 2) % NBUF)
        s_wait(NCH - 1, (NCH - 1) % IRING, (NCH - 1) % NBUF)

        plsc.subcore_barrier()

        @pl.when(sid < NS - 1)
        def _():
            @pl.loop(0, ROWS_A // 208)
            def _(b):
                r0 = sid * ROWS_A + b * 208
                pltpu.sync_copy(acc_sh.at[pl.ds(r0, 208)],
                                out_hbm.at[cid].at[pl.ds(r0, 208)])

        @pl.when(sid == NS - 1)
        def _():
            @pl.loop(0, ROWS_B // 160)
            def _(b):
                r0 = (NS - 1) * ROWS_A + b * 160
                pltpu.sync_copy(acc_sh.at[pl.ds(r0, 160)],
                                out_hbm.at[cid].at[pl.ds(r0, 160)])

    return k(h, idx)


_BLK = 10000


def _linear_tc(x, W, b):
    def body(x_ref, w_ref, b_ref, o_ref):
        o_ref[...] = lax.dot_general(
            x_ref[...], w_ref[...], (((1,), (1,)), ((), ())),
            preferred_element_type=jnp.float32) + b_ref[...]

    return pl.pallas_call(
        body,
        grid=(N // _BLK,),
        in_specs=[pl.BlockSpec((_BLK, D), lambda i: (i, 0)),
                  pl.BlockSpec((D, D), lambda i: (0, 0)),
                  pl.BlockSpec((1, D), lambda i: (0, 0))],
        out_specs=pl.BlockSpec((_BLK, D), lambda i: (i, 0)),
        out_shape=jax.ShapeDtypeStruct((N, D), jnp.float32),
    )(x, W, b.reshape(1, D))


def _elu_linear_tc(p, W, b):
    def body(p_ref, w_ref, b_ref, o_ref):
        s = p_ref[0] + p_ref[1]
        z = jnp.where(s > 0, s, jnp.exp(s) - 1.0)
        o_ref[...] = lax.dot_general(
            z, w_ref[...], (((1,), (1,)), ((), ())),
            preferred_element_type=jnp.float32) + b_ref[...]

    return pl.pallas_call(
        body,
        grid=(N // _BLK,),
        in_specs=[pl.BlockSpec((NC, _BLK, D), lambda i: (0, i, 0)),
                  pl.BlockSpec((D, D), lambda i: (0, 0)),
                  pl.BlockSpec((1, D), lambda i: (0, 0))],
        out_specs=pl.BlockSpec((_BLK, D), lambda i: (i, 0)),
        out_shape=jax.ShapeDtypeStruct((N, D), jnp.float32),
    )(p, W, b.reshape(1, D))


def _sum2_tc(q):
    def body(q_ref, o_ref):
        o_ref[...] = q_ref[0] + q_ref[1]

    return pl.pallas_call(
        body,
        grid=(N // _BLK,),
        in_specs=[pl.BlockSpec((NC, _BLK, D), lambda i: (0, i, 0))],
        out_specs=pl.BlockSpec((_BLK, D), lambda i: (i, 0)),
        out_shape=jax.ShapeDtypeStruct((N, D), jnp.float32),
    )(q)


def kernel(x, edge_index, edge_weight, W1, b1, W2, b2):
    pad = E_PAD - E
    pad_idx = (jnp.arange(pad, dtype=jnp.int32) * 37) % N
    src = jnp.concatenate([edge_index[1], pad_idx])
    dst = jnp.concatenate([edge_index[0], pad_idx])
    w = jnp.concatenate([edge_weight, jnp.zeros((pad,), jnp.float32)])
    idx = jnp.stack([src.reshape(NW, NCH, CHUNK),
                     dst.reshape(NW, NCH, CHUNK),
                     lax.bitcast_convert_type(w, jnp.int32).reshape(
                         NW, NCH, CHUNK)], axis=2)

    h1 = _linear_tc(x, W1, b1)
    p = _spmm_sc(h1, idx)
    h2 = _elu_linear_tc(p, W2, b2)
    q = _spmm_sc(h2, idx)
    return _sum2_tc(q)
